# Initial kernel scaffold; baseline (speedup 1.0000x reference)
import jax
import jax.numpy as jnp
from jax.experimental import pallas as pl


def kernel(x_author, x_paper, edge_index_writes, edge_index_rev, edge_index_cites, batch_author, batch_paper, Wl0_writes, bl0_writes, Wr0_writes, Wl0_rev, bl0_rev, Wr0_rev, Wl0_cites, bl0_cites, Wr0_cites, Wl1_writes, bl1_writes, Wr1_writes, Wl1_rev, bl1_rev, Wr1_rev, Wl1_cites, bl1_cites, Wr1_cites, W_mlp, b_mlp, W_lin, b_lin):
    raise NotImplementedError("write your pallas kernel here")


# speedup vs baseline: 2.1035x; 2.1035x over previous
import functools

import jax
import jax.numpy as jnp
from jax import lax
from jax.experimental import pallas as pl
from jax.experimental.pallas import tpu as pltpu
from jax.experimental.pallas import tpu_sc as plsc

N = 10000
E = 160000
D = 256
HALF = 128
NG = 64
NT = 16
EPT = E // NT
C = 80
NCHUNK = EPT // C
RCH = 80
NRC = N // RCH
RQ = (NRC + NT - 1) // NT


def _seg_body(with_deg, *refs):
    xa, xp, esw, edw, esc, edc, esr, edr = refs[:8]
    rest = refs[8:]
    if with_deg:
        s_w, s_c, s_r, degs = rest[:4]
        rest = rest[4:]
    else:
        s_w, s_c, s_r = rest[:3]
        degs = None
        rest = rest[3:]
    acc, sidx, didx, rows, sem = rest

    c = lax.axis_index("c")
    t = lax.axis_index("s")
    xoff = c * N

    def fill_rows(val):
        def _f(k, carry):
            rows[k // 8, pl.ds((k % 8) * 16, 16)] = jnp.full((16,), val, jnp.float32)
            return carry
        lax.fori_loop(0, RCH * 8, _f, 0)

    def zero_acc():
        fill_rows(0.0)
        for q in range(RQ):
            k = q * NT + t
            @pl.when(k < NRC)
            def _():
                pltpu.sync_copy(rows, acc.at[pl.ds(k * RCH, RCH)])

    def drain_acc(out_ref, slot, off):
        for q in range(RQ):
            k = q * NT + t
            @pl.when(k < NRC)
            def _():
                pltpu.sync_copy(acc.at[pl.ds(k * RCH, RCH)], rows)
                if slot is None:
                    pltpu.sync_copy(rows, out_ref.at[pl.ds(off + k * RCH, RCH)])
                else:
                    pltpu.sync_copy(rows, out_ref.at[slot, pl.ds(k * RCH, RCH)])

    def phase(x_ref, es_ref, ed_ref, out_ref):
        zero_acc()
        plsc.subcore_barrier()

        def chunk_body(i, carry):
            base = t * EPT + i * C
            pltpu.sync_copy(es_ref.at[pl.ds(base, C)], sidx)
            pltpu.sync_copy(ed_ref.at[pl.ds(base, C)], didx)
            for j in range(C // 16):
                sidx[pl.ds(j * 16, 16)] = sidx[pl.ds(j * 16, 16)] + xoff
            pltpu.async_copy(x_ref.at[sidx], rows, sem).wait()
            pltpu.sync_copy(rows, acc.at[didx], add=True)
            return carry
        lax.fori_loop(0, NCHUNK, chunk_body, 0)
        plsc.subcore_barrier()
        drain_acc(out_ref, None, xoff)
        plsc.subcore_barrier()

    def phase_deg(ed_ref, slot):
        zero_acc()
        fill_rows(1.0)
        plsc.subcore_barrier()

        def chunk_body(i, carry):
            base = t * EPT + i * C
            pltpu.sync_copy(ed_ref.at[pl.ds(base, C)], didx)
            pltpu.sync_copy(rows, acc.at[didx], add=True)
            return carry
        lax.fori_loop(0, NCHUNK, chunk_body, 0)
        plsc.subcore_barrier()
        drain_acc(degs, slot, 0)
        plsc.subcore_barrier()

    phase(xa, esw, edw, s_w)
    phase(xp, esc, edc, s_c)
    phase(xp, esr, edr, s_r)
    if with_deg:
        @pl.when(c == 0)
        def _():
            phase_deg(edw, 0)

        @pl.when(c == 1)
        def _():
            phase_deg(edc, 1)

        @pl.when(c == 0)
        def _():
            phase_deg(edr, 2)


def _make_seg_kernel(with_deg):
    outs = [jax.ShapeDtypeStruct((2 * N, HALF), jnp.float32) for _ in range(3)]
    if with_deg:
        outs.append(jax.ShapeDtypeStruct((3, N, HALF), jnp.float32))
    scratch = [
        pltpu.VMEM_SHARED((N, HALF), jnp.float32),
        pltpu.VMEM((C,), jnp.int32),
        pltpu.VMEM((C,), jnp.int32),
        pltpu.VMEM((C, HALF), jnp.float32),
        pltpu.SemaphoreType.DMA,
    ]
    mesh = plsc.VectorSubcoreMesh(core_axis_name="c", subcore_axis_name="s",
                                  num_cores=2, num_subcores=NT)
    return pl.kernel(
        functools.partial(_seg_body, with_deg),
        out_type=tuple(outs),
        mesh=mesh,
        scratch_types=scratch,
    )


_seg_l0 = _make_seg_kernel(True)
_seg_l1 = _make_seg_kernel(False)

R = 1000
GRID = N // R


def _paper_body(sw, dw, sc_, dc, xp, wlw, wlc, wrw, wrc, bw, bc, out):
    cdims = (((1,), (1,)), ((), ()))
    mw = sw[...] / jnp.maximum(dw[...], 1.0)
    mc = sc_[...] / jnp.maximum(dc[...], 1.0)
    o = lax.dot_general(mw, wlw[...], cdims, preferred_element_type=jnp.float32)
    o += lax.dot_general(mc, wlc[...], cdims, preferred_element_type=jnp.float32)
    o += lax.dot_general(xp[...], wrw[...] + wrc[...], cdims,
                         preferred_element_type=jnp.float32)
    o += bw[...] + bc[...]
    out[...] = jnp.where(o >= 0, o, 0.01 * o)


def _author_body(sr, dr, xa, wlr, wrr, br, out):
    cdims = (((1,), (1,)), ((), ()))
    mr = sr[...] / jnp.maximum(dr[...], 1.0)
    o = lax.dot_general(mr, wlr[...], cdims, preferred_element_type=jnp.float32)
    o += lax.dot_general(xa[...], wrr[...], cdims, preferred_element_type=jnp.float32)
    o += br[...]
    out[...] = jnp.where(o >= 0, o, 0.01 * o)


def _row_spec():
    return pl.BlockSpec((R, D), lambda i: (i, 0))


def _deg_spec():
    return pl.BlockSpec((R, 1), lambda i: (i, 0))


def _full_spec(shape):
    nd = len(shape)
    return pl.BlockSpec(shape, lambda i: (0,) * nd)


_paper_tc = pl.pallas_call(
    _paper_body,
    grid=(GRID,),
    in_specs=[
        _row_spec(), _deg_spec(), _row_spec(), _deg_spec(), _row_spec(),
        _full_spec((D, D)), _full_spec((D, D)), _full_spec((D, D)),
        _full_spec((D, D)), _full_spec((1, D)), _full_spec((1, D)),
    ],
    out_specs=_row_spec(),
    out_shape=jax.ShapeDtypeStruct((N, D), jnp.float32),
)

_author_tc = pl.pallas_call(
    _author_body,
    grid=(GRID,),
    in_specs=[
        _row_spec(), _deg_spec(), _row_spec(),
        _full_spec((D, D)), _full_spec((D, D)), _full_spec((1, D)),
    ],
    out_specs=_row_spec(),
    out_shape=jax.ShapeDtypeStruct((N, D), jnp.float32),
)


def _pool_body(xa, xp, ba, bp, out):
    i = pl.program_id(0)
    iot = lax.broadcasted_iota(jnp.int32, (NG, R), 0)
    oh_a = (iot == ba[0]).astype(jnp.float32)
    oh_p = (iot == bp[0]).astype(jnp.float32)
    pa = lax.dot_general(oh_a, xa[...], (((1,), (0,)), ((), ())),
                         preferred_element_type=jnp.float32)
    pp = lax.dot_general(oh_p, xp[...], (((1,), (0,)), ((), ())),
                         preferred_element_type=jnp.float32)

    @pl.when(i == 0)
    def _():
        out[:, 0:D] = pa
        out[:, D:2 * D] = pp

    @pl.when(i > 0)
    def _():
        out[:, 0:D] += pa
        out[:, D:2 * D] += pp


_pool_tc = pl.pallas_call(
    _pool_body,
    grid=(GRID,),
    in_specs=[
        _row_spec(), _row_spec(),
        pl.BlockSpec((1, 1, R), lambda i: (i, 0, 0)),
        pl.BlockSpec((1, 1, R), lambda i: (i, 0, 0)),
    ],
    out_specs=pl.BlockSpec((NG, 2 * D), lambda i: (0, 0)),
    out_shape=jax.ShapeDtypeStruct((NG, 2 * D), jnp.float32),
)


def _head_body(rep, wm, bm, wl, bl, out):
    cdims = (((1,), (1,)), ((), ()))
    h = lax.dot_general(rep[...], wm[...], cdims,
                        preferred_element_type=jnp.float32) + bm[...]
    out[...] = lax.dot_general(h, wl[...], cdims,
                               preferred_element_type=jnp.float32) + bl[...]


_head_tc = pl.pallas_call(
    _head_body,
    out_shape=jax.ShapeDtypeStruct((NG, 128), jnp.float32),
)


def _flat_half(x):
    return jnp.concatenate([x[:, :HALF], x[:, HALF:]], axis=0)


def _unflat(sf):
    return jnp.concatenate([sf[:N], sf[N:]], axis=1)


def kernel(x_author, x_paper, edge_index_writes, edge_index_rev, edge_index_cites,
           batch_author, batch_paper,
           Wl0_writes, bl0_writes, Wr0_writes,
           Wl0_rev, bl0_rev, Wr0_rev,
           Wl0_cites, bl0_cites, Wr0_cites,
           Wl1_writes, bl1_writes, Wr1_writes,
           Wl1_rev, bl1_rev, Wr1_rev,
           Wl1_cites, bl1_cites, Wr1_cites,
           W_mlp, b_mlp, W_lin, b_lin):
    f32 = jnp.float32
    xa = x_author.astype(f32)
    xp = x_paper.astype(f32)
    ei_w = edge_index_writes.astype(jnp.int32)
    ei_r = edge_index_rev.astype(jnp.int32)
    ei_c = edge_index_cites.astype(jnp.int32)

    edges = (ei_w[0], ei_w[1], ei_c[0], ei_c[1], ei_r[0], ei_r[1])

    sw_f, sc_f, sr_f, degs = _seg_l0(_flat_half(xa), _flat_half(xp), *edges)
    dw = degs[0, :, 0:1]
    dc = degs[1, :, 0:1]
    dr = degs[2, :, 0:1]

    b = lambda v: v.reshape(1, -1).astype(f32)
    xp1 = _paper_tc(_unflat(sw_f), dw, _unflat(sc_f), dc, xp,
                    Wl0_writes, Wl0_cites, Wr0_writes, Wr0_cites,
                    b(bl0_writes), b(bl0_cites))
    xa1 = _author_tc(_unflat(sr_f), dr, xa, Wl0_rev, Wr0_rev, b(bl0_rev))

    sw_f, sc_f, sr_f = _seg_l1(_flat_half(xa1), _flat_half(xp1), *edges)
    xp2 = _paper_tc(_unflat(sw_f), dw, _unflat(sc_f), dc, xp1,
                    Wl1_writes, Wl1_cites, Wr1_writes, Wr1_cites,
                    b(bl1_writes), b(bl1_cites))
    xa2 = _author_tc(_unflat(sr_f), dr, xa1, Wl1_rev, Wr1_rev, b(bl1_rev))

    ba = batch_author.astype(jnp.int32).reshape(GRID, 1, R)
    bp = batch_paper.astype(jnp.int32).reshape(GRID, 1, R)
    rep = _pool_tc(xa2, xp2, ba, bp)
    return _head_tc(rep, W_mlp, b(b_mlp), W_lin, b(b_lin))


# speedup vs baseline: 3.6790x; 1.7490x over previous
import functools

import jax
import jax.numpy as jnp
from jax import lax
from jax.experimental import pallas as pl
from jax.experimental.pallas import tpu as pltpu
from jax.experimental.pallas import tpu_sc as plsc

N = 10000
E = 160000
D = 256
HALF = 128
NG = 64
NT = 16
EPT = E // NT
C = 80
NCHUNK = EPT // C
RCH = 80
NRC = N // RCH
RQ = (NRC + NT - 1) // NT


def _seg_body(with_deg, *refs):
    xa, xp, cw, cc, cr = refs[:5]
    rest = refs[5:]
    if with_deg:
        s_w, s_c, s_r, degs = rest[:4]
        rest = rest[4:]
    else:
        s_w, s_c, s_r = rest[:3]
        degs = None
        rest = rest[3:]
    acc = rest[0]
    cidx = list(rest[1:4])
    sidx = list(rest[4:7])
    didx = list(rest[7:10])
    rows = list(rest[10:13])
    semi, semg, sems = rest[13:16]

    c = lax.axis_index("c")
    t = lax.axis_index("s")
    xoff = c * N

    def fill_rows0(val):
        def _f(k, carry):
            rows[0][k // 8, pl.ds((k % 8) * 16, 16)] = jnp.full(
                (16,), val, jnp.float32)
            return carry
        lax.fori_loop(0, RCH * 8, _f, 0)

    def zero_acc():
        fill_rows0(0.0)
        for q in range(RQ):
            k = q * NT + t
            @pl.when(k < NRC)
            def _():
                pltpu.sync_copy(rows[0], acc.at[pl.ds(k * RCH, RCH)])

    def drain_acc(out_ref, slot, off):
        for q in range(RQ):
            k = q * NT + t
            @pl.when(k < NRC)
            def _():
                pltpu.sync_copy(acc.at[pl.ds(k * RCH, RCH)], rows[0])
                if slot is None:
                    pltpu.sync_copy(rows[0], out_ref.at[pl.ds(off + k * RCH, RCH)])
                else:
                    pltpu.sync_copy(rows[0], out_ref.at[slot, pl.ds(k * RCH, RCH)])

    def run_phase(comb_ref, out_ref, slot, x_ref):
        gather = x_ref is not None
        zero_acc()
        if not gather:
            fill_rows0(1.0)
        plsc.subcore_barrier()
        tbase = t * (2 * EPT)

        def load_cidx(jj, p):
            pltpu.async_copy(comb_ref.at[pl.ds(tbase + jj * 2 * C, 2 * C)],
                             cidx[p], semi)

        def build(jj, p):
            pltpu.make_async_copy(
                comb_ref.at[pl.ds(tbase + jj * 2 * C, 2 * C)],
                cidx[p], semi).wait()
            for k in range(C // 16):
                if gather:
                    sidx[p][pl.ds(k * 16, 16)] = (
                        cidx[p][pl.ds(k * 16, 16)] + xoff)
                didx[p][pl.ds(k * 16, 16)] = cidx[p][pl.ds(C + k * 16, 16)]

        def gath(p):
            pltpu.async_copy(x_ref.at[sidx[p]], rows[p], semg)

        def wait_gath(p):
            pltpu.make_async_copy(x_ref.at[sidx[p]], rows[p], semg).wait()

        def scat(p):
            src = rows[p] if gather else rows[0]
            pltpu.async_copy(src, acc.at[didx[p]], sems, add=True)

        def wait_scat(p):
            src = rows[p] if gather else rows[0]
            pltpu.make_async_copy(src, acc.at[didx[p]], sems).wait()

        load_cidx(0, 0)
        load_cidx(1, 1)
        build(0, 0)
        if gather:
            gath(0)

        def body(j, carry):
            pm = lax.rem(j, 3)
            for pb in range(3):
                @pl.when(pm == pb)
                def _():
                    p1 = (pb + 1) % 3
                    p2 = (pb + 2) % 3
                    if gather:
                        wait_gath(pb)
                    scat(pb)

                    @pl.when(j >= 2)
                    def _():
                        wait_scat(p1)

                    @pl.when(j + 2 < NCHUNK)
                    def _():
                        load_cidx(j + 2, p2)

                    @pl.when(j + 1 < NCHUNK)
                    def _():
                        build(j + 1, p1)
                        if gather:
                            gath(p1)
            return carry
        lax.fori_loop(0, NCHUNK, body, 0)
        wait_scat((NCHUNK - 2) % 3)
        wait_scat((NCHUNK - 1) % 3)
        plsc.subcore_barrier()
        if slot is None:
            drain_acc(out_ref, None, xoff)
        else:
            drain_acc(out_ref, slot, 0)
        plsc.subcore_barrier()

    run_phase(cw, s_w, None, xa)
    run_phase(cc, s_c, None, xp)
    run_phase(cr, s_r, None, xp)
    if with_deg:
        @pl.when(c == 0)
        def _():
            run_phase(cw, degs, 0, None)

        @pl.when(c == 1)
        def _():
            run_phase(cc, degs, 1, None)

        @pl.when(c == 0)
        def _():
            run_phase(cr, degs, 2, None)


def _make_seg_kernel(with_deg):
    outs = [jax.ShapeDtypeStruct((2 * N, HALF), jnp.float32) for _ in range(3)]
    if with_deg:
        outs.append(jax.ShapeDtypeStruct((3, N, HALF), jnp.float32))
    scratch = (
        [pltpu.VMEM_SHARED((N, HALF), jnp.float32)]
        + [pltpu.VMEM((2 * C,), jnp.int32)] * 3
        + [pltpu.VMEM((C,), jnp.int32)] * 3
        + [pltpu.VMEM((C,), jnp.int32)] * 3
        + [pltpu.VMEM((C, HALF), jnp.float32)] * 3
        + [pltpu.SemaphoreType.DMA] * 3
    )
    mesh = plsc.VectorSubcoreMesh(core_axis_name="c", subcore_axis_name="s",
                                  num_cores=2, num_subcores=NT)
    return pl.kernel(
        functools.partial(_seg_body, with_deg),
        out_type=tuple(outs),
        mesh=mesh,
        scratch_types=scratch,
    )


_seg_l0 = _make_seg_kernel(True)
_seg_l1 = _make_seg_kernel(False)

R = 1000
GRID = N // R


def _paper_body(sw, dw, sc_, dc, xp, wlw, wlc, wrw, wrc, bw, bc, out):
    cdims = (((1,), (1,)), ((), ()))
    mw = sw[...] / jnp.maximum(dw[...], 1.0)
    mc = sc_[...] / jnp.maximum(dc[...], 1.0)
    o = lax.dot_general(mw, wlw[...], cdims, preferred_element_type=jnp.float32)
    o += lax.dot_general(mc, wlc[...], cdims, preferred_element_type=jnp.float32)
    o += lax.dot_general(xp[...], wrw[...] + wrc[...], cdims,
                         preferred_element_type=jnp.float32)
    o += bw[...] + bc[...]
    out[...] = jnp.where(o >= 0, o, 0.01 * o)


def _author_body(sr, dr, xa, wlr, wrr, br, out):
    cdims = (((1,), (1,)), ((), ()))
    mr = sr[...] / jnp.maximum(dr[...], 1.0)
    o = lax.dot_general(mr, wlr[...], cdims, preferred_element_type=jnp.float32)
    o += lax.dot_general(xa[...], wrr[...], cdims, preferred_element_type=jnp.float32)
    o += br[...]
    out[...] = jnp.where(o >= 0, o, 0.01 * o)


def _row_spec():
    return pl.BlockSpec((R, D), lambda i: (i, 0))


def _deg_spec():
    return pl.BlockSpec((R, 1), lambda i: (i, 0))


def _full_spec(shape):
    nd = len(shape)
    return pl.BlockSpec(shape, lambda i: (0,) * nd)


_paper_tc = pl.pallas_call(
    _paper_body,
    grid=(GRID,),
    in_specs=[
        _row_spec(), _deg_spec(), _row_spec(), _deg_spec(), _row_spec(),
        _full_spec((D, D)), _full_spec((D, D)), _full_spec((D, D)),
        _full_spec((D, D)), _full_spec((1, D)), _full_spec((1, D)),
    ],
    out_specs=_row_spec(),
    out_shape=jax.ShapeDtypeStruct((N, D), jnp.float32),
)

_author_tc = pl.pallas_call(
    _author_body,
    grid=(GRID,),
    in_specs=[
        _row_spec(), _deg_spec(), _row_spec(),
        _full_spec((D, D)), _full_spec((D, D)), _full_spec((1, D)),
    ],
    out_specs=_row_spec(),
    out_shape=jax.ShapeDtypeStruct((N, D), jnp.float32),
)


def _pool_body(xa, xp, ba, bp, out):
    i = pl.program_id(0)
    iot = lax.broadcasted_iota(jnp.int32, (NG, R), 0)
    oh_a = (iot == ba[0]).astype(jnp.float32)
    oh_p = (iot == bp[0]).astype(jnp.float32)
    pa = lax.dot_general(oh_a, xa[...], (((1,), (0,)), ((), ())),
                         preferred_element_type=jnp.float32)
    pp = lax.dot_general(oh_p, xp[...], (((1,), (0,)), ((), ())),
                         preferred_element_type=jnp.float32)

    @pl.when(i == 0)
    def _():
        out[:, 0:D] = pa
        out[:, D:2 * D] = pp

    @pl.when(i > 0)
    def _():
        out[:, 0:D] += pa
        out[:, D:2 * D] += pp


_pool_tc = pl.pallas_call(
    _pool_body,
    grid=(GRID,),
    in_specs=[
        _row_spec(), _row_spec(),
        pl.BlockSpec((1, 1, R), lambda i: (i, 0, 0)),
        pl.BlockSpec((1, 1, R), lambda i: (i, 0, 0)),
    ],
    out_specs=pl.BlockSpec((NG, 2 * D), lambda i: (0, 0)),
    out_shape=jax.ShapeDtypeStruct((NG, 2 * D), jnp.float32),
)


def _head_body(rep, wm, bm, wl, bl, out):
    cdims = (((1,), (1,)), ((), ()))
    h = lax.dot_general(rep[...], wm[...], cdims,
                        preferred_element_type=jnp.float32) + bm[...]
    out[...] = lax.dot_general(h, wl[...], cdims,
                               preferred_element_type=jnp.float32) + bl[...]


_head_tc = pl.pallas_call(
    _head_body,
    out_shape=jax.ShapeDtypeStruct((NG, 128), jnp.float32),
)


def _flat_half(x):
    return jnp.concatenate([x[:, :HALF], x[:, HALF:]], axis=0)


def _unflat(sf):
    return jnp.concatenate([sf[:N], sf[N:]], axis=1)


def kernel(x_author, x_paper, edge_index_writes, edge_index_rev, edge_index_cites,
           batch_author, batch_paper,
           Wl0_writes, bl0_writes, Wr0_writes,
           Wl0_rev, bl0_rev, Wr0_rev,
           Wl0_cites, bl0_cites, Wr0_cites,
           Wl1_writes, bl1_writes, Wr1_writes,
           Wl1_rev, bl1_rev, Wr1_rev,
           Wl1_cites, bl1_cites, Wr1_cites,
           W_mlp, b_mlp, W_lin, b_lin):
    f32 = jnp.float32
    xa = x_author.astype(f32)
    xp = x_paper.astype(f32)
    ei_w = edge_index_writes.astype(jnp.int32)
    ei_r = edge_index_rev.astype(jnp.int32)
    ei_c = edge_index_cites.astype(jnp.int32)

    def comb(ei):
        s2 = ei[0].reshape(NT, NCHUNK, 1, C)
        d2 = ei[1].reshape(NT, NCHUNK, 1, C)
        return jnp.concatenate([s2, d2], axis=2).reshape(-1)

    edges = (comb(ei_w), comb(ei_c), comb(ei_r))

    sw_f, sc_f, sr_f, degs = _seg_l0(_flat_half(xa), _flat_half(xp), *edges)
    dw = degs[0, :, 0:1]
    dc = degs[1, :, 0:1]
    dr = degs[2, :, 0:1]

    b = lambda v: v.reshape(1, -1).astype(f32)
    xp1 = _paper_tc(_unflat(sw_f), dw, _unflat(sc_f), dc, xp,
                    Wl0_writes, Wl0_cites, Wr0_writes, Wr0_cites,
                    b(bl0_writes), b(bl0_cites))
    xa1 = _author_tc(_unflat(sr_f), dr, xa, Wl0_rev, Wr0_rev, b(bl0_rev))

    sw_f, sc_f, sr_f = _seg_l1(_flat_half(xa1), _flat_half(xp1), *edges)
    xp2 = _paper_tc(_unflat(sw_f), dw, _unflat(sc_f), dc, xp1,
                    Wl1_writes, Wl1_cites, Wr1_writes, Wr1_cites,
                    b(bl1_writes), b(bl1_cites))
    xa2 = _author_tc(_unflat(sr_f), dr, xa1, Wl1_rev, Wr1_rev, b(bl1_rev))

    ba = batch_author.astype(jnp.int32).reshape(GRID, 1, R)
    bp = batch_paper.astype(jnp.int32).reshape(GRID, 1, R)
    rep = _pool_tc(xa2, xp2, ba, bp)
    return _head_tc(rep, W_mlp, b(b_mlp), W_lin, b(b_lin))


# speedup vs baseline: 3.6905x; 1.0031x over previous
import functools

import jax
import jax.numpy as jnp
from jax import lax
from jax.experimental import pallas as pl
from jax.experimental.pallas import tpu as pltpu
from jax.experimental.pallas import tpu_sc as plsc

N = 10000
E = 160000
D = 256
HALF = 128
NG = 64
NT = 16
EPT = E // NT
C = 80
NCHUNK = EPT // C
RCH = 80
NRC = N // RCH
RQ = (NRC + NT - 1) // NT


def _seg_body(with_deg, *refs):
    xa, xp, cw, cc, cr = refs[:5]
    rest = refs[5:]
    if with_deg:
        s_w, s_c, s_r, degs = rest[:4]
        rest = rest[4:]
    else:
        s_w, s_c, s_r = rest[:3]
        degs = None
        rest = rest[3:]
    acc = rest[0]
    cidx = list(rest[1:4])
    sidx = list(rest[4:7])
    didx = list(rest[7:10])
    rows = list(rest[10:13])
    semi, semg, sems = rest[13:16]

    c = lax.axis_index("c")
    t = lax.axis_index("s")
    xoff = c * N

    def fill_rows0(val):
        def _f(k, carry):
            rows[0][k // 8, pl.ds((k % 8) * 16, 16)] = jnp.full(
                (16,), val, jnp.float32)
            return carry
        lax.fori_loop(0, RCH * 8, _f, 0)

    def zero_acc():
        fill_rows0(0.0)
        for q in range(RQ):
            k = q * NT + t
            @pl.when(k < NRC)
            def _():
                pltpu.sync_copy(rows[0], acc.at[pl.ds(k * RCH, RCH)])

    def drain_acc(out_ref, slot, off):
        for q in range(RQ):
            k = q * NT + t
            @pl.when(k < NRC)
            def _():
                pltpu.sync_copy(acc.at[pl.ds(k * RCH, RCH)], rows[0])
                if slot is None:
                    pltpu.sync_copy(rows[0], out_ref.at[pl.ds(off + k * RCH, RCH)])
                else:
                    pltpu.sync_copy(rows[0], out_ref.at[slot, pl.ds(k * RCH, RCH)])

    def run_phase(comb_ref, out_ref, slot, x_ref):
        gather = x_ref is not None
        zero_acc()
        if not gather:
            fill_rows0(1.0)
        plsc.subcore_barrier()
        tbase = t * (2 * EPT)

        def load_cidx(jj, p):
            pltpu.async_copy(comb_ref.at[pl.ds(tbase + jj * 2 * C, 2 * C)],
                             cidx[p], semi)

        def build(jj, p):
            pltpu.make_async_copy(
                comb_ref.at[pl.ds(tbase + jj * 2 * C, 2 * C)],
                cidx[p], semi).wait()
            for k in range(C // 16):
                if gather:
                    sidx[p][pl.ds(k * 16, 16)] = (
                        cidx[p][pl.ds(k * 16, 16)] + xoff)
                didx[p][pl.ds(k * 16, 16)] = cidx[p][pl.ds(C + k * 16, 16)]

        def gath(p):
            pltpu.async_copy(x_ref.at[sidx[p]], rows[p], semg)

        def wait_gath(p):
            pltpu.make_async_copy(x_ref.at[sidx[p]], rows[p], semg).wait()

        def scat(p):
            src = rows[p] if gather else rows[0]
            pltpu.async_copy(src, acc.at[didx[p]], sems, add=True)

        def wait_scat(p):
            src = rows[p] if gather else rows[0]
            pltpu.make_async_copy(src, acc.at[didx[p]], sems).wait()

        load_cidx(0, 0)
        load_cidx(1, 1)
        build(0, 0)
        if gather:
            gath(0)

        def emit_iter(j, jj):
            p = j % 3
            p1 = (p + 1) % 3
            p2 = (p + 2) % 3
            if gather:
                wait_gath(p)
            scat(p)
            if j >= 2:
                wait_scat(p1)
            if j + 2 < NCHUNK:
                load_cidx(jj + 2, p2)
            if j + 1 < NCHUNK:
                build(jj + 1, p1)
                if gather:
                    gath(p1)

        STEADY = (NCHUNK - 4) // 3
        TAIL = 3 * STEADY + 2
        emit_iter(0, 0)
        emit_iter(1, 1)

        def body(k, carry):
            j = 2 + 3 * k
            emit_iter(2, j)
            emit_iter(3, j + 1)
            emit_iter(4, j + 2)
            return carry
        lax.fori_loop(0, STEADY, body, 0)
        for j in range(TAIL, NCHUNK):
            emit_iter(j, j)
        wait_scat((NCHUNK - 2) % 3)
        wait_scat((NCHUNK - 1) % 3)
        plsc.subcore_barrier()
        if slot is None:
            drain_acc(out_ref, None, xoff)
        else:
            drain_acc(out_ref, slot, 0)
        plsc.subcore_barrier()

    run_phase(cw, s_w, None, xa)
    run_phase(cc, s_c, None, xp)
    run_phase(cr, s_r, None, xp)
    if with_deg:
        @pl.when(c == 0)
        def _():
            run_phase(cw, degs, 0, None)

        @pl.when(c == 1)
        def _():
            run_phase(cc, degs, 1, None)

        @pl.when(c == 0)
        def _():
            run_phase(cr, degs, 2, None)


def _make_seg_kernel(with_deg):
    outs = [jax.ShapeDtypeStruct((2 * N, HALF), jnp.float32) for _ in range(3)]
    if with_deg:
        outs.append(jax.ShapeDtypeStruct((3, N, HALF), jnp.float32))
    scratch = (
        [pltpu.VMEM_SHARED((N, HALF), jnp.float32)]
        + [pltpu.VMEM((2 * C,), jnp.int32)] * 3
        + [pltpu.VMEM((C,), jnp.int32)] * 3
        + [pltpu.VMEM((C,), jnp.int32)] * 3
        + [pltpu.VMEM((C, HALF), jnp.float32)] * 3
        + [pltpu.SemaphoreType.DMA] * 3
    )
    mesh = plsc.VectorSubcoreMesh(core_axis_name="c", subcore_axis_name="s",
                                  num_cores=2, num_subcores=NT)
    return pl.kernel(
        functools.partial(_seg_body, with_deg),
        out_type=tuple(outs),
        mesh=mesh,
        scratch_types=scratch,
    )


_seg_l0 = _make_seg_kernel(True)
_seg_l1 = _make_seg_kernel(False)

R = 1000
GRID = N // R


def _paper_body(sw, dw, sc_, dc, xp, wlw, wlc, wrw, wrc, bw, bc, out):
    cdims = (((1,), (1,)), ((), ()))
    mw = sw[...] / jnp.maximum(dw[...], 1.0)
    mc = sc_[...] / jnp.maximum(dc[...], 1.0)
    o = lax.dot_general(mw, wlw[...], cdims, preferred_element_type=jnp.float32)
    o += lax.dot_general(mc, wlc[...], cdims, preferred_element_type=jnp.float32)
    o += lax.dot_general(xp[...], wrw[...] + wrc[...], cdims,
                         preferred_element_type=jnp.float32)
    o += bw[...] + bc[...]
    out[...] = jnp.where(o >= 0, o, 0.01 * o)


def _author_body(sr, dr, xa, wlr, wrr, br, out):
    cdims = (((1,), (1,)), ((), ()))
    mr = sr[...] / jnp.maximum(dr[...], 1.0)
    o = lax.dot_general(mr, wlr[...], cdims, preferred_element_type=jnp.float32)
    o += lax.dot_general(xa[...], wrr[...], cdims, preferred_element_type=jnp.float32)
    o += br[...]
    out[...] = jnp.where(o >= 0, o, 0.01 * o)


def _row_spec():
    return pl.BlockSpec((R, D), lambda i: (i, 0))


def _deg_spec():
    return pl.BlockSpec((R, 1), lambda i: (i, 0))


def _full_spec(shape):
    nd = len(shape)
    return pl.BlockSpec(shape, lambda i: (0,) * nd)


_paper_tc = pl.pallas_call(
    _paper_body,
    grid=(GRID,),
    in_specs=[
        _row_spec(), _deg_spec(), _row_spec(), _deg_spec(), _row_spec(),
        _full_spec((D, D)), _full_spec((D, D)), _full_spec((D, D)),
        _full_spec((D, D)), _full_spec((1, D)), _full_spec((1, D)),
    ],
    out_specs=_row_spec(),
    out_shape=jax.ShapeDtypeStruct((N, D), jnp.float32),
)

_author_tc = pl.pallas_call(
    _author_body,
    grid=(GRID,),
    in_specs=[
        _row_spec(), _deg_spec(), _row_spec(),
        _full_spec((D, D)), _full_spec((D, D)), _full_spec((1, D)),
    ],
    out_specs=_row_spec(),
    out_shape=jax.ShapeDtypeStruct((N, D), jnp.float32),
)


def _pool_body(xa, xp, ba, bp, out):
    i = pl.program_id(0)
    iot = lax.broadcasted_iota(jnp.int32, (NG, R), 0)
    oh_a = (iot == ba[0]).astype(jnp.float32)
    oh_p = (iot == bp[0]).astype(jnp.float32)
    pa = lax.dot_general(oh_a, xa[...], (((1,), (0,)), ((), ())),
                         preferred_element_type=jnp.float32)
    pp = lax.dot_general(oh_p, xp[...], (((1,), (0,)), ((), ())),
                         preferred_element_type=jnp.float32)

    @pl.when(i == 0)
    def _():
        out[:, 0:D] = pa
        out[:, D:2 * D] = pp

    @pl.when(i > 0)
    def _():
        out[:, 0:D] += pa
        out[:, D:2 * D] += pp


_pool_tc = pl.pallas_call(
    _pool_body,
    grid=(GRID,),
    in_specs=[
        _row_spec(), _row_spec(),
        pl.BlockSpec((1, 1, R), lambda i: (i, 0, 0)),
        pl.BlockSpec((1, 1, R), lambda i: (i, 0, 0)),
    ],
    out_specs=pl.BlockSpec((NG, 2 * D), lambda i: (0, 0)),
    out_shape=jax.ShapeDtypeStruct((NG, 2 * D), jnp.float32),
)


def _head_body(rep, wm, bm, wl, bl, out):
    cdims = (((1,), (1,)), ((), ()))
    h = lax.dot_general(rep[...], wm[...], cdims,
                        preferred_element_type=jnp.float32) + bm[...]
    out[...] = lax.dot_general(h, wl[...], cdims,
                               preferred_element_type=jnp.float32) + bl[...]


_head_tc = pl.pallas_call(
    _head_body,
    out_shape=jax.ShapeDtypeStruct((NG, 128), jnp.float32),
)


def _flat_half(x):
    return jnp.concatenate([x[:, :HALF], x[:, HALF:]], axis=0)


def _unflat(sf):
    return jnp.concatenate([sf[:N], sf[N:]], axis=1)


def kernel(x_author, x_paper, edge_index_writes, edge_index_rev, edge_index_cites,
           batch_author, batch_paper,
           Wl0_writes, bl0_writes, Wr0_writes,
           Wl0_rev, bl0_rev, Wr0_rev,
           Wl0_cites, bl0_cites, Wr0_cites,
           Wl1_writes, bl1_writes, Wr1_writes,
           Wl1_rev, bl1_rev, Wr1_rev,
           Wl1_cites, bl1_cites, Wr1_cites,
           W_mlp, b_mlp, W_lin, b_lin):
    f32 = jnp.float32
    xa = x_author.astype(f32)
    xp = x_paper.astype(f32)
    ei_w = edge_index_writes.astype(jnp.int32)
    ei_r = edge_index_rev.astype(jnp.int32)
    ei_c = edge_index_cites.astype(jnp.int32)

    def comb(ei):
        s2 = ei[0].reshape(NT, NCHUNK, 1, C)
        d2 = ei[1].reshape(NT, NCHUNK, 1, C)
        return jnp.concatenate([s2, d2], axis=2).reshape(-1)

    edges = (comb(ei_w), comb(ei_c), comb(ei_r))

    sw_f, sc_f, sr_f, degs = _seg_l0(_flat_half(xa), _flat_half(xp), *edges)
    dw = degs[0, :, 0:1]
    dc = degs[1, :, 0:1]
    dr = degs[2, :, 0:1]

    b = lambda v: v.reshape(1, -1).astype(f32)
    xp1 = _paper_tc(_unflat(sw_f), dw, _unflat(sc_f), dc, xp,
                    Wl0_writes, Wl0_cites, Wr0_writes, Wr0_cites,
                    b(bl0_writes), b(bl0_cites))
    xa1 = _author_tc(_unflat(sr_f), dr, xa, Wl0_rev, Wr0_rev, b(bl0_rev))

    sw_f, sc_f, sr_f = _seg_l1(_flat_half(xa1), _flat_half(xp1), *edges)
    xp2 = _paper_tc(_unflat(sw_f), dw, _unflat(sc_f), dc, xp1,
                    Wl1_writes, Wl1_cites, Wr1_writes, Wr1_cites,
                    b(bl1_writes), b(bl1_cites))
    xa2 = _author_tc(_unflat(sr_f), dr, xa1, Wl1_rev, Wr1_rev, b(bl1_rev))

    ba = batch_author.astype(jnp.int32).reshape(GRID, 1, R)
    bp = batch_paper.astype(jnp.int32).reshape(GRID, 1, R)
    rep = _pool_tc(xa2, xp2, ba, bp)
    return _head_tc(rep, W_mlp, b(b_mlp), W_lin, b(b_lin))
